# Initial kernel scaffold; baseline (speedup 1.0000x reference)
import jax
import jax.numpy as jnp
from jax.experimental import pallas as pl


def kernel(features, edge_index, W1, b1, W2, b2, W3, b3):
    raise NotImplementedError("write your pallas kernel here")


# speedup vs baseline: 14.6624x; 14.6624x over previous
import dataclasses
import functools

import jax
import jax.numpy as jnp
from jax import lax
from jax.experimental import pallas as pl
from jax.experimental.pallas import tpu as pltpu
from jax.experimental.pallas import tpu_sc as plsc

N = 10000
E = 320000
D = 128

NC = 2
NS = 16
NW = NC * NS

NACC = 10496
RPT = NACC // NS
CHUNK = 128
CPT = 80
EPT = CPT * CHUNK
EPAD = NW * EPT
PPT = EPT - E // NW
ZR = 164

_MESH = dict(core_axis_name="c", subcore_axis_name="s")

_SC_PARAMS = pltpu.CompilerParams(use_tc_tiling_on_sc=False)
if "needs_layout_passes" in pltpu.CompilerParams.__dataclass_fields__:
    _SC_PARAMS = dataclasses.replace(_SC_PARAMS, needs_layout_passes=False)


def _rsqrt_inplace(nbuf, rows):
    @pl.loop(0, rows, step=16)
    def _(q):
        x = jnp.maximum(nbuf[pl.ds(q, 16)], 1.0)
        xi = plsc.bitcast(x, jnp.int32)
        yi = jnp.full((16,), 0x5F3759DF, jnp.int32) - lax.shift_right_logical(
            xi, jnp.full((16,), 1, jnp.int32))
        y = plsc.bitcast(yi, jnp.float32)
        for _ in range(3):
            y = y * (1.5 - 0.5 * x * y * y)
        nbuf[pl.ds(q, 16)] = y


def _deg_norms(srcp, dstp):
    mesh = plsc.VectorSubcoreMesh(**_MESH)

    @functools.partial(
        pl.kernel,
        out_type=(jax.ShapeDtypeStruct((NACC,), jnp.float32),
                  jax.ShapeDtypeStruct((NACC,), jnp.float32)),
        mesh=mesh,
        scratch_types=[
            pltpu.VMEM((CPT, CHUNK), jnp.int32),
            pltpu.VMEM((CHUNK,), jnp.float32),
            pltpu.VMEM((RPT,), jnp.float32),
            pltpu.VMEM_SHARED((NACC,), jnp.float32),
            pltpu.VMEM_SHARED((NACC,), jnp.float32),
        ],
        compiler_params=_SC_PARAMS,
    )
    def k(srcp_hbm, dstp_hbm, ns_hbm, nd_hbm, idxb, ones, nbuf, dego, degi):
        c = lax.axis_index("c")
        s = lax.axis_index("s")

        @pl.when(c == 0)
        def _():
            @pl.loop(0, CHUNK, step=16)
            def _(q):
                ones[pl.ds(q, 16)] = jnp.ones((16,), jnp.float32)
            @pl.loop(0, RPT, step=16)
            def _(q):
                nbuf[pl.ds(q, 16)] = jnp.zeros((16,), jnp.float32)
            base = s * RPT
            pltpu.sync_copy(nbuf, dego.at[pl.ds(base, RPT)])
            pltpu.sync_copy(nbuf, degi.at[pl.ds(base, RPT)])
            plsc.subcore_barrier()
            for half in range(2):
                w = half * NS + s
                pltpu.sync_copy(srcp_hbm.at[w], idxb)
                @pl.loop(0, CPT)
                def _(j):
                    pltpu.sync_copy(ones, dego.at[idxb.at[j]], add=True)
                pltpu.sync_copy(dstp_hbm.at[w], idxb)
                @pl.loop(0, CPT)
                def _(j):
                    pltpu.sync_copy(ones, degi.at[idxb.at[j]], add=True)
            plsc.subcore_barrier()
            pltpu.sync_copy(dego.at[pl.ds(base, RPT)], nbuf)
            _rsqrt_inplace(nbuf, RPT)
            pltpu.sync_copy(nbuf, ns_hbm.at[pl.ds(base, RPT)])
            pltpu.sync_copy(degi.at[pl.ds(base, RPT)], nbuf)
            _rsqrt_inplace(nbuf, RPT)
            pltpu.sync_copy(nbuf, nd_hbm.at[pl.ds(base, RPT)])

    return k(srcp, dstp)


DH = D // 2


def _make_spmm():
    mesh = plsc.VectorSubcoreMesh(**_MESH)

    @functools.partial(
        pl.kernel,
        out_type=(jax.ShapeDtypeStruct((NC, NACC, DH), jnp.float32),
                  jax.ShapeDtypeStruct((NC, NACC, DH), jnp.float32)),
        mesh=mesh,
        scratch_types=[
            pltpu.VMEM((CPT, CHUNK), jnp.int32),
            pltpu.VMEM((CPT, CHUNK), jnp.int32),
            pltpu.VMEM((CHUNK, DH), jnp.float32),
            pltpu.VMEM((CHUNK, DH), jnp.float32),
            pltpu.VMEM((ZR, DH), jnp.float32),
            pltpu.VMEM_SHARED((NACC, DH), jnp.float32),
            pltpu.SemaphoreType.DMA,
            pltpu.SemaphoreType.DMA,
        ],
        compiler_params=_SC_PARAMS,
        name="gcn_spmm",
    )
    def k(hw0_hbm, hw1_hbm, srcp_hbm, dstp_hbm, out0_hbm, out1_hbm,
          sidx, didx, bufA, bufB, zb, acc, semA, semB):
        c = lax.axis_index("c")
        s = lax.axis_index("s")
        w = c * NS + s
        base = s * RPT

        @pl.loop(0, ZR)
        def _(r):
            @pl.loop(0, DH, step=16)
            def _(q):
                zb[r, pl.ds(q, 16)] = jnp.zeros((16,), jnp.float32)

        pltpu.sync_copy(srcp_hbm.at[w], sidx)
        pltpu.sync_copy(dstp_hbm.at[w], didx)
        for t in range(RPT // ZR):
            pltpu.sync_copy(zb, acc.at[pl.ds(base + t * ZR, ZR)])
        plsc.subcore_barrier()

        for hw_hbm, out_hbm in ((hw0_hbm, out0_hbm), (hw1_hbm, out1_hbm)):
            pltpu.async_copy(hw_hbm.at[sidx.at[0]], bufA, semA)
            pltpu.async_copy(hw_hbm.at[sidx.at[1]], bufB, semB)

            @pl.loop(0, CPT, step=2)
            def _(j):
                pltpu.make_async_copy(hw_hbm.at[sidx.at[j]], bufA, semA).wait()
                pltpu.sync_copy(bufA, acc.at[didx.at[j]], add=True)
                @pl.when(j + 2 < CPT)
                def _():
                    pltpu.async_copy(hw_hbm.at[sidx.at[j + 2]], bufA, semA)
                pltpu.make_async_copy(
                    hw_hbm.at[sidx.at[j + 1]], bufB, semB).wait()
                pltpu.sync_copy(bufB, acc.at[didx.at[j + 1]], add=True)
                @pl.when(j + 3 < CPT)
                def _():
                    pltpu.async_copy(hw_hbm.at[sidx.at[j + 3]], bufB, semB)

            plsc.subcore_barrier()
            pltpu.sync_copy(acc.at[pl.ds(base, RPT)],
                            out_hbm.at[c, pl.ds(base, RPT)])
            for t in range(RPT // ZR):
                pltpu.sync_copy(zb, acc.at[pl.ds(base + t * ZR, ZR)])
            plsc.subcore_barrier()

    return k


_spmm = _make_spmm()


def _write_halves(o0_ref, o1_ref, hw):
    o0_ref[pl.ds(0, N), :] = hw[:, 0:DH]
    o1_ref[pl.ds(0, N), :] = hw[:, DH:D]
    zpad = jnp.zeros((NACC - N, DH), jnp.float32)
    o0_ref[pl.ds(N, NACC - N), :] = zpad
    o1_ref[pl.ds(N, NACC - N), :] = zpad


def _sum_planes(a0_ref, a1_ref):
    lo = a0_ref[0, pl.ds(0, N), :] + a0_ref[1, pl.ds(0, N), :]
    hi = a1_ref[0, pl.ds(0, N), :] + a1_ref[1, pl.ds(0, N), :]
    return jnp.concatenate([lo, hi], axis=1)


def _tc_stage_a(x, W, ns_col):
    def body(x_ref, w_ref, ns_ref, o0_ref, o1_ref):
        xw = jnp.dot(x_ref[...], w_ref[...], preferred_element_type=jnp.float32)
        _write_halves(o0_ref, o1_ref, xw * ns_ref[...])

    return pl.pallas_call(
        body, out_shape=(jax.ShapeDtypeStruct((NACC, DH), jnp.float32),
                         jax.ShapeDtypeStruct((NACC, DH), jnp.float32)),
    )(x, W, ns_col)


def _tc_stage_b(agg0, agg1, nd_col, b, W, ns_col):
    def body(a0_ref, a1_ref, nd_ref, b_ref, w_ref, ns_ref, o0_ref, o1_ref):
        h = jnp.maximum(_sum_planes(a0_ref, a1_ref) * nd_ref[...] + b_ref[...],
                        0.0)
        hw = jnp.dot(h, w_ref[...], preferred_element_type=jnp.float32)
        _write_halves(o0_ref, o1_ref, hw * ns_ref[...])

    return pl.pallas_call(
        body, out_shape=(jax.ShapeDtypeStruct((NACC, DH), jnp.float32),
                         jax.ShapeDtypeStruct((NACC, DH), jnp.float32)),
    )(agg0, agg1, nd_col, b, W, ns_col)


def _tc_stage_c(agg0, agg1, nd_col, b2, W3, b3):
    def body(a0_ref, a1_ref, nd_ref, b2_ref, w_ref, b3_ref, o_ref):
        h = jnp.maximum(
            _sum_planes(a0_ref, a1_ref) * nd_ref[...] + b2_ref[...], 0.0)
        o_ref[...] = jnp.dot(
            h, w_ref[...], preferred_element_type=jnp.float32) + b3_ref[...]

    return pl.pallas_call(
        body, out_shape=jax.ShapeDtypeStruct((N, D), jnp.float32),
    )(agg0, agg1, nd_col, b2, W3, b3)


def kernel(features, edge_index, W1, b1, W2, b2, W3, b3):
    src = edge_index[0]
    dst = edge_index[1]
    padv = (N + jnp.arange(NW * PPT, dtype=jnp.int32) % (NACC - N)).reshape(
        NW, PPT)
    srcp = jnp.concatenate(
        [src.reshape(NW, E // NW), padv], axis=1).reshape(NW, CPT, CHUNK)
    dstp = jnp.concatenate(
        [dst.reshape(NW, E // NW), padv], axis=1).reshape(NW, CPT, CHUNK)

    ns_vec, nd_vec = _deg_norms(srcp, dstp)
    ns_col = ns_vec[:N].reshape(N, 1)
    nd_col = nd_vec[:N].reshape(N, 1)

    hw1a, hw1b = _tc_stage_a(features, W1, ns_col)
    agg1a, agg1b = _spmm(hw1a, hw1b, srcp, dstp)
    hw2a, hw2b = _tc_stage_b(agg1a, agg1b, nd_col, b1.reshape(1, D), W2, ns_col)
    agg2a, agg2b = _spmm(hw2a, hw2b, srcp, dstp)
    return _tc_stage_c(agg2a, agg2b, nd_col, b2.reshape(1, D), W3,
                       b3.reshape(1, D))


# speedup vs baseline: 17.0923x; 1.1657x over previous
import dataclasses
import functools

import jax
import jax.numpy as jnp
from jax import lax
from jax.experimental import pallas as pl
from jax.experimental.pallas import tpu as pltpu
from jax.experimental.pallas import tpu_sc as plsc

N = 10000
E = 320000
D = 128

NC = 2
NS = 16
NW = NC * NS

NACC = 10496
RPT = NACC // NS
CHUNK = 128
CPT = 80
EPT = CPT * CHUNK
EPAD = NW * EPT
PPT = EPT - E // NW
ZR = 82
NBUF = 4

_MESH = dict(core_axis_name="c", subcore_axis_name="s")

_SC_PARAMS = pltpu.CompilerParams(use_tc_tiling_on_sc=False)
if "needs_layout_passes" in pltpu.CompilerParams.__dataclass_fields__:
    _SC_PARAMS = dataclasses.replace(_SC_PARAMS, needs_layout_passes=False)


def _rsqrt_inplace(nbuf, rows):
    @pl.loop(0, rows, step=16)
    def _(q):
        x = jnp.maximum(nbuf[pl.ds(q, 16)], 1.0)
        xi = plsc.bitcast(x, jnp.int32)
        yi = jnp.full((16,), 0x5F3759DF, jnp.int32) - lax.shift_right_logical(
            xi, jnp.full((16,), 1, jnp.int32))
        y = plsc.bitcast(yi, jnp.float32)
        for _ in range(3):
            y = y * (1.5 - 0.5 * x * y * y)
        nbuf[pl.ds(q, 16)] = y


def _deg_norms(srcp, dstp):
    mesh = plsc.VectorSubcoreMesh(**_MESH)

    @functools.partial(
        pl.kernel,
        out_type=(jax.ShapeDtypeStruct((NACC,), jnp.float32),
                  jax.ShapeDtypeStruct((NACC,), jnp.float32)),
        mesh=mesh,
        scratch_types=[
            pltpu.VMEM((CPT, CHUNK), jnp.int32),
            pltpu.VMEM((CHUNK,), jnp.float32),
            pltpu.VMEM((RPT,), jnp.float32),
            pltpu.VMEM_SHARED((NACC,), jnp.float32),
            pltpu.VMEM_SHARED((NACC,), jnp.float32),
        ],
        compiler_params=_SC_PARAMS,
    )
    def k(srcp_hbm, dstp_hbm, ns_hbm, nd_hbm, idxb, ones, nbuf, dego, degi):
        c = lax.axis_index("c")
        s = lax.axis_index("s")

        @pl.when(c == 0)
        def _():
            @pl.loop(0, CHUNK, step=16)
            def _(q):
                ones[pl.ds(q, 16)] = jnp.ones((16,), jnp.float32)
            @pl.loop(0, RPT, step=16)
            def _(q):
                nbuf[pl.ds(q, 16)] = jnp.zeros((16,), jnp.float32)
            base = s * RPT
            pltpu.sync_copy(nbuf, dego.at[pl.ds(base, RPT)])
            pltpu.sync_copy(nbuf, degi.at[pl.ds(base, RPT)])
            plsc.subcore_barrier()
            for half in range(2):
                w = half * NS + s
                pltpu.sync_copy(srcp_hbm.at[w], idxb)
                @pl.loop(0, CPT)
                def _(j):
                    pltpu.sync_copy(ones, dego.at[idxb.at[j]], add=True)
                pltpu.sync_copy(dstp_hbm.at[w], idxb)
                @pl.loop(0, CPT)
                def _(j):
                    pltpu.sync_copy(ones, degi.at[idxb.at[j]], add=True)
            plsc.subcore_barrier()
            pltpu.sync_copy(dego.at[pl.ds(base, RPT)], nbuf)
            _rsqrt_inplace(nbuf, RPT)
            pltpu.sync_copy(nbuf, ns_hbm.at[pl.ds(base, RPT)])
            pltpu.sync_copy(degi.at[pl.ds(base, RPT)], nbuf)
            _rsqrt_inplace(nbuf, RPT)
            pltpu.sync_copy(nbuf, nd_hbm.at[pl.ds(base, RPT)])

    return k(srcp, dstp)


DH = D // 2


def _make_spmm():
    mesh = plsc.VectorSubcoreMesh(**_MESH)

    @functools.partial(
        pl.kernel,
        out_type=(jax.ShapeDtypeStruct((NC, NACC, DH), jnp.float32),
                  jax.ShapeDtypeStruct((NC, NACC, DH), jnp.float32)),
        mesh=mesh,
        scratch_types=[
            pltpu.VMEM((CPT, CHUNK), jnp.int32),
            pltpu.VMEM((CPT, CHUNK), jnp.int32),
        ] + [pltpu.VMEM((CHUNK, DH), jnp.float32) for _ in range(NBUF)] + [
            pltpu.VMEM((ZR, DH), jnp.float32),
            pltpu.VMEM_SHARED((NACC, DH), jnp.float32),
        ] + [pltpu.SemaphoreType.DMA for _ in range(NBUF)],
        compiler_params=_SC_PARAMS,
        name="gcn_spmm",
    )
    def k(hw0_hbm, hw1_hbm, srcp_hbm, dstp_hbm, out0_hbm, out1_hbm,
          sidx, didx, *rest):
        bufs = rest[:NBUF]
        zb = rest[NBUF]
        acc = rest[NBUF + 1]
        sems = rest[NBUF + 2:]
        c = lax.axis_index("c")
        s = lax.axis_index("s")
        w = c * NS + s
        base = s * RPT

        @pl.loop(0, ZR)
        def _(r):
            @pl.loop(0, DH, step=16)
            def _(q):
                zb[r, pl.ds(q, 16)] = jnp.zeros((16,), jnp.float32)

        pltpu.sync_copy(srcp_hbm.at[w], sidx)
        pltpu.sync_copy(dstp_hbm.at[w], didx)
        for t in range(RPT // ZR):
            pltpu.sync_copy(zb, acc.at[pl.ds(base + t * ZR, ZR)])
        plsc.subcore_barrier()

        for hw_hbm, out_hbm in ((hw0_hbm, out0_hbm), (hw1_hbm, out1_hbm)):
            for b in range(NBUF):
                pltpu.async_copy(hw_hbm.at[sidx.at[b]], bufs[b], sems[b])

            @pl.loop(0, CPT, step=NBUF)
            def _(j):
                for b in range(NBUF):
                    pltpu.make_async_copy(
                        hw_hbm.at[sidx.at[j + b]], bufs[b], sems[b]).wait()
                    pltpu.sync_copy(bufs[b], acc.at[didx.at[j + b]], add=True)
                    @pl.when(j + b + NBUF < CPT)
                    def _():
                        pltpu.async_copy(
                            hw_hbm.at[sidx.at[j + b + NBUF]], bufs[b], sems[b])

            plsc.subcore_barrier()
            pltpu.sync_copy(acc.at[pl.ds(base, RPT)],
                            out_hbm.at[c, pl.ds(base, RPT)])
            for t in range(RPT // ZR):
                pltpu.sync_copy(zb, acc.at[pl.ds(base + t * ZR, ZR)])
            plsc.subcore_barrier()

    return k


_spmm = _make_spmm()


def _write_halves(o0_ref, o1_ref, hw):
    o0_ref[pl.ds(0, N), :] = hw[:, 0:DH]
    o1_ref[pl.ds(0, N), :] = hw[:, DH:D]
    zpad = jnp.zeros((NACC - N, DH), jnp.float32)
    o0_ref[pl.ds(N, NACC - N), :] = zpad
    o1_ref[pl.ds(N, NACC - N), :] = zpad


def _sum_planes(a0_ref, a1_ref):
    lo = a0_ref[0, pl.ds(0, N), :] + a0_ref[1, pl.ds(0, N), :]
    hi = a1_ref[0, pl.ds(0, N), :] + a1_ref[1, pl.ds(0, N), :]
    return jnp.concatenate([lo, hi], axis=1)


def _tc_stage_a(x, W, ns_col):
    def body(x_ref, w_ref, ns_ref, o0_ref, o1_ref):
        xw = jnp.dot(x_ref[...], w_ref[...], preferred_element_type=jnp.float32)
        _write_halves(o0_ref, o1_ref, xw * ns_ref[...])

    return pl.pallas_call(
        body, out_shape=(jax.ShapeDtypeStruct((NACC, DH), jnp.float32),
                         jax.ShapeDtypeStruct((NACC, DH), jnp.float32)),
    )(x, W, ns_col)


def _tc_stage_b(agg0, agg1, nd_col, b, W, ns_col):
    def body(a0_ref, a1_ref, nd_ref, b_ref, w_ref, ns_ref, o0_ref, o1_ref):
        h = jnp.maximum(_sum_planes(a0_ref, a1_ref) * nd_ref[...] + b_ref[...],
                        0.0)
        hw = jnp.dot(h, w_ref[...], preferred_element_type=jnp.float32)
        _write_halves(o0_ref, o1_ref, hw * ns_ref[...])

    return pl.pallas_call(
        body, out_shape=(jax.ShapeDtypeStruct((NACC, DH), jnp.float32),
                         jax.ShapeDtypeStruct((NACC, DH), jnp.float32)),
    )(agg0, agg1, nd_col, b, W, ns_col)


def _tc_stage_c(agg0, agg1, nd_col, b2, W3, b3):
    def body(a0_ref, a1_ref, nd_ref, b2_ref, w_ref, b3_ref, o_ref):
        h = jnp.maximum(
            _sum_planes(a0_ref, a1_ref) * nd_ref[...] + b2_ref[...], 0.0)
        o_ref[...] = jnp.dot(
            h, w_ref[...], preferred_element_type=jnp.float32) + b3_ref[...]

    return pl.pallas_call(
        body, out_shape=jax.ShapeDtypeStruct((N, D), jnp.float32),
    )(agg0, agg1, nd_col, b2, W3, b3)


def kernel(features, edge_index, W1, b1, W2, b2, W3, b3):
    src = edge_index[0]
    dst = edge_index[1]
    padv = (N + jnp.arange(NW * PPT, dtype=jnp.int32) % (NACC - N)).reshape(
        NW, PPT)
    srcp = jnp.concatenate(
        [src.reshape(NW, E // NW), padv], axis=1).reshape(NW, CPT, CHUNK)
    dstp = jnp.concatenate(
        [dst.reshape(NW, E // NW), padv], axis=1).reshape(NW, CPT, CHUNK)

    ns_vec, nd_vec = _deg_norms(srcp, dstp)
    ns_col = ns_vec[:N].reshape(N, 1)
    nd_col = nd_vec[:N].reshape(N, 1)

    hw1a, hw1b = _tc_stage_a(features, W1, ns_col)
    agg1a, agg1b = _spmm(hw1a, hw1b, srcp, dstp)
    hw2a, hw2b = _tc_stage_b(agg1a, agg1b, nd_col, b1.reshape(1, D), W2, ns_col)
    agg2a, agg2b = _spmm(hw2a, hw2b, srcp, dstp)
    return _tc_stage_c(agg2a, agg2b, nd_col, b2.reshape(1, D), W3,
                       b3.reshape(1, D))


# speedup vs baseline: 18.0439x; 1.0557x over previous
import dataclasses
import functools

import jax
import jax.numpy as jnp
from jax import lax
from jax.experimental import pallas as pl
from jax.experimental.pallas import tpu as pltpu
from jax.experimental.pallas import tpu_sc as plsc

N = 10000
E = 320000
D = 128

NC = 2
NS = 16
NW = NC * NS

NACC = 10496
RPT = NACC // NS
CHUNK = 128
CPT = 80
EPT = CPT * CHUNK
EPAD = NW * EPT
PPT = EPT - E // NW
ZR = 82
NBUF = 4

_MESH = dict(core_axis_name="c", subcore_axis_name="s")

_SC_PARAMS = pltpu.CompilerParams(use_tc_tiling_on_sc=False)
if "needs_layout_passes" in pltpu.CompilerParams.__dataclass_fields__:
    _SC_PARAMS = dataclasses.replace(_SC_PARAMS, needs_layout_passes=False)


def _rsqrt_inplace(nbuf, rows):
    @pl.loop(0, rows, step=16)
    def _(q):
        x = jnp.maximum(nbuf[pl.ds(q, 16)], 1.0)
        xi = plsc.bitcast(x, jnp.int32)
        yi = jnp.full((16,), 0x5F3759DF, jnp.int32) - lax.shift_right_logical(
            xi, jnp.full((16,), 1, jnp.int32))
        y = plsc.bitcast(yi, jnp.float32)
        for _ in range(3):
            y = y * (1.5 - 0.5 * x * y * y)
        nbuf[pl.ds(q, 16)] = y


def _deg_norms(srcp, dstp):
    mesh = plsc.VectorSubcoreMesh(**_MESH)

    @functools.partial(
        pl.kernel,
        out_type=(jax.ShapeDtypeStruct((NACC,), jnp.float32),
                  jax.ShapeDtypeStruct((NACC,), jnp.float32)),
        mesh=mesh,
        scratch_types=[
            pltpu.VMEM((CPT, CHUNK), jnp.int32),
            pltpu.VMEM((CHUNK,), jnp.float32),
            pltpu.VMEM((RPT,), jnp.float32),
            pltpu.VMEM_SHARED((NACC,), jnp.float32),
        ],
        compiler_params=_SC_PARAMS,
    )
    def k(srcp_hbm, dstp_hbm, ns_hbm, nd_hbm, idxb, ones, nbuf, deg):
        c = lax.axis_index("c")
        s = lax.axis_index("s")

        @pl.loop(0, CHUNK, step=16)
        def _(q):
            ones[pl.ds(q, 16)] = jnp.ones((16,), jnp.float32)
        @pl.loop(0, RPT, step=16)
        def _(q):
            nbuf[pl.ds(q, 16)] = jnp.zeros((16,), jnp.float32)
        base = s * RPT
        pltpu.sync_copy(nbuf, deg.at[pl.ds(base, RPT)])
        plsc.subcore_barrier()
        for half in range(2):
            w = half * NS + s
            @pl.when(c == 0)
            def _():
                pltpu.sync_copy(srcp_hbm.at[w], idxb)
            @pl.when(c == 1)
            def _():
                pltpu.sync_copy(dstp_hbm.at[w], idxb)
            @pl.loop(0, CPT)
            def _(j):
                pltpu.sync_copy(ones, deg.at[idxb.at[j]], add=True)
        plsc.subcore_barrier()
        pltpu.sync_copy(deg.at[pl.ds(base, RPT)], nbuf)
        _rsqrt_inplace(nbuf, RPT)
        @pl.when(c == 0)
        def _():
            pltpu.sync_copy(nbuf, ns_hbm.at[pl.ds(base, RPT)])
        @pl.when(c == 1)
        def _():
            pltpu.sync_copy(nbuf, nd_hbm.at[pl.ds(base, RPT)])

    return k(srcp, dstp)


DH = D // 2


def _make_spmm():
    mesh = plsc.VectorSubcoreMesh(**_MESH)

    @functools.partial(
        pl.kernel,
        out_type=(jax.ShapeDtypeStruct((NC, NACC, DH), jnp.float32),
                  jax.ShapeDtypeStruct((NC, NACC, DH), jnp.float32)),
        mesh=mesh,
        scratch_types=[
            pltpu.VMEM((CPT, CHUNK), jnp.int32),
            pltpu.VMEM((CPT, CHUNK), jnp.int32),
        ] + [pltpu.VMEM((CHUNK, DH), jnp.float32) for _ in range(NBUF)] + [
            pltpu.VMEM((ZR, DH), jnp.float32),
            pltpu.VMEM_SHARED((NACC, DH), jnp.float32),
        ] + [pltpu.SemaphoreType.DMA for _ in range(NBUF)],
        compiler_params=_SC_PARAMS,
        name="gcn_spmm",
    )
    def k(hw0_hbm, hw1_hbm, srcp_hbm, dstp_hbm, out0_hbm, out1_hbm,
          sidx, didx, *rest):
        bufs = rest[:NBUF]
        zb = rest[NBUF]
        acc = rest[NBUF + 1]
        sems = rest[NBUF + 2:]
        c = lax.axis_index("c")
        s = lax.axis_index("s")
        w = c * NS + s
        base = s * RPT

        @pl.loop(0, ZR)
        def _(r):
            @pl.loop(0, DH, step=16)
            def _(q):
                zb[r, pl.ds(q, 16)] = jnp.zeros((16,), jnp.float32)

        pltpu.sync_copy(srcp_hbm.at[w], sidx)
        pltpu.sync_copy(dstp_hbm.at[w], didx)
        for t in range(RPT // ZR):
            pltpu.sync_copy(zb, acc.at[pl.ds(base + t * ZR, ZR)])
        plsc.subcore_barrier()

        for hw_hbm, out_hbm in ((hw0_hbm, out0_hbm), (hw1_hbm, out1_hbm)):
            for b in range(NBUF):
                pltpu.async_copy(hw_hbm.at[sidx.at[b]], bufs[b], sems[b])

            @pl.loop(0, CPT, step=NBUF)
            def _(j):
                for b in range(NBUF):
                    pltpu.make_async_copy(
                        hw_hbm.at[sidx.at[j + b]], bufs[b], sems[b]).wait()
                    pltpu.sync_copy(bufs[b], acc.at[didx.at[j + b]], add=True)
                    @pl.when(j + b + NBUF < CPT)
                    def _():
                        pltpu.async_copy(
                            hw_hbm.at[sidx.at[j + b + NBUF]], bufs[b], sems[b])

            plsc.subcore_barrier()
            pltpu.sync_copy(acc.at[pl.ds(base, RPT)],
                            out_hbm.at[c, pl.ds(base, RPT)])
            for t in range(RPT // ZR):
                pltpu.sync_copy(zb, acc.at[pl.ds(base + t * ZR, ZR)])
            plsc.subcore_barrier()

    return k


_spmm = _make_spmm()


def _write_halves(o0_ref, o1_ref, hw):
    o0_ref[pl.ds(0, N), :] = hw[:, 0:DH]
    o1_ref[pl.ds(0, N), :] = hw[:, DH:D]
    zpad = jnp.zeros((NACC - N, DH), jnp.float32)
    o0_ref[pl.ds(N, NACC - N), :] = zpad
    o1_ref[pl.ds(N, NACC - N), :] = zpad


def _sum_planes(a0_ref, a1_ref):
    lo = a0_ref[0, pl.ds(0, N), :] + a0_ref[1, pl.ds(0, N), :]
    hi = a1_ref[0, pl.ds(0, N), :] + a1_ref[1, pl.ds(0, N), :]
    return jnp.concatenate([lo, hi], axis=1)


def _tc_stage_a(x, W, ns_col):
    def body(x_ref, w_ref, ns_ref, o0_ref, o1_ref):
        xw = jnp.dot(x_ref[...], w_ref[...], preferred_element_type=jnp.float32)
        _write_halves(o0_ref, o1_ref, xw * ns_ref[...])

    return pl.pallas_call(
        body, out_shape=(jax.ShapeDtypeStruct((NACC, DH), jnp.float32),
                         jax.ShapeDtypeStruct((NACC, DH), jnp.float32)),
    )(x, W, ns_col)


def _tc_stage_b(agg0, agg1, nd_col, b, W, ns_col):
    def body(a0_ref, a1_ref, nd_ref, b_ref, w_ref, ns_ref, o0_ref, o1_ref):
        h = jnp.maximum(_sum_planes(a0_ref, a1_ref) * nd_ref[...] + b_ref[...],
                        0.0)
        hw = jnp.dot(h, w_ref[...], preferred_element_type=jnp.float32)
        _write_halves(o0_ref, o1_ref, hw * ns_ref[...])

    return pl.pallas_call(
        body, out_shape=(jax.ShapeDtypeStruct((NACC, DH), jnp.float32),
                         jax.ShapeDtypeStruct((NACC, DH), jnp.float32)),
    )(agg0, agg1, nd_col, b, W, ns_col)


def _tc_stage_c(agg0, agg1, nd_col, b2, W3, b3):
    def body(a0_ref, a1_ref, nd_ref, b2_ref, w_ref, b3_ref, o_ref):
        h = jnp.maximum(
            _sum_planes(a0_ref, a1_ref) * nd_ref[...] + b2_ref[...], 0.0)
        o_ref[...] = jnp.dot(
            h, w_ref[...], preferred_element_type=jnp.float32) + b3_ref[...]

    return pl.pallas_call(
        body, out_shape=jax.ShapeDtypeStruct((N, D), jnp.float32),
    )(agg0, agg1, nd_col, b2, W3, b3)


def kernel(features, edge_index, W1, b1, W2, b2, W3, b3):
    src = edge_index[0]
    dst = edge_index[1]
    padv = (N + jnp.arange(NW * PPT, dtype=jnp.int32) % (NACC - N)).reshape(
        NW, PPT)
    srcp = jnp.concatenate(
        [src.reshape(NW, E // NW), padv], axis=1).reshape(NW, CPT, CHUNK)
    dstp = jnp.concatenate(
        [dst.reshape(NW, E // NW), padv], axis=1).reshape(NW, CPT, CHUNK)

    ns_vec, nd_vec = _deg_norms(srcp, dstp)
    ns_col = ns_vec[:N].reshape(N, 1)
    nd_col = nd_vec[:N].reshape(N, 1)

    hw1a, hw1b = _tc_stage_a(features, W1, ns_col)
    agg1a, agg1b = _spmm(hw1a, hw1b, srcp, dstp)
    hw2a, hw2b = _tc_stage_b(agg1a, agg1b, nd_col, b1.reshape(1, D), W2, ns_col)
    agg2a, agg2b = _spmm(hw2a, hw2b, srcp, dstp)
    return _tc_stage_c(agg2a, agg2b, nd_col, b2.reshape(1, D), W3,
                       b3.reshape(1, D))
